# Initial kernel scaffold; baseline (speedup 1.0000x reference)
import jax
import jax.numpy as jnp
from jax.experimental import pallas as pl


def kernel(edges, node_features, edge_features, params):
    raise NotImplementedError("write your pallas kernel here")


# speedup vs baseline: 7.6533x; 7.6533x over previous
import functools

import jax
import jax.numpy as jnp
from jax import lax
from jax.experimental import pallas as pl
from jax.experimental.pallas import tpu as pltpu
from jax.experimental.pallas import tpu_sc as plsc

N = 10000
E = 320000
D_NODE = 128
D_EDGE = 16
W = 64

GW = 128
EBLK = 2000
NSUB = 16
NCORE = 2
RPT = N // NSUB


def _f32dot(x, w):
    return jnp.dot(x, w, preferred_element_type=jnp.float32)


def _mlp_refs(x, r):
    h = jnp.maximum(_f32dot(x, r[0][...]) + r[1][...], 0.0)
    h1 = jnp.maximum(_f32dot(h, r[2][...]) + r[3][...], 0.0)
    h2 = jnp.maximum(_f32dot(h1, r[4][...]) + r[5][...], 0.0)
    h = h + h2
    return _f32dot(h, r[6][...]) + r[7][...]


def _res_out(h, r):
    h1 = jnp.maximum(_f32dot(h, r[0][...]) + r[1][...], 0.0)
    h2 = jnp.maximum(_f32dot(h1, r[2][...]) + r[3][...], 0.0)
    h = h + h2
    return _f32dot(h, r[4][...]) + r[5][...]


def _mlp_params(p):
    return [p["in"]["W"], p["in"]["b"].reshape(1, -1),
            p["blocks"][0]["a"]["W"], p["blocks"][0]["a"]["b"].reshape(1, -1),
            p["blocks"][0]["b"]["W"], p["blocks"][0]["b"]["b"].reshape(1, -1),
            p["out"]["W"], p["out"]["b"].reshape(1, -1)]


def _k1_body(x_ref, *refs):
    o_ref = refs[-1]
    o_ref[...] = _mlp_refs(x_ref[...], refs[:-1])


def _tc_node_encoder(nf, ws):
    return pl.pallas_call(
        _k1_body,
        out_shape=jax.ShapeDtypeStruct((N, W), jnp.float32),
    )(nf, *ws)


def _sc_gather(n_enc, e):
    mesh = plsc.VectorSubcoreMesh(core_axis_name="c", subcore_axis_name="s")

    @functools.partial(
        pl.kernel,
        out_type=jax.ShapeDtypeStruct((2, E, W), jnp.float32),
        mesh=mesh,
        compiler_params=pltpu.CompilerParams(use_tc_tiling_on_sc=False),
    )
    def k(tab_hbm, idx_hbm, out_hbm):
        def body(i_vmem, o_vmem):
            pltpu.sync_copy(tab_hbm.at[i_vmem.at[0]], o_vmem.at[0])

        pltpu.emit_pipeline(
            body,
            grid=(2, E // GW),
            in_specs=[pl.BlockSpec((1, GW), lambda j, i: (j, i))],
            out_specs=[pl.BlockSpec((1, GW, W), lambda j, i: (j, i, 0))],
            core_axis_name=("c", "s"),
            dimension_semantics=(pltpu.PARALLEL, pltpu.PARALLEL),
        )(idx_hbm, out_hbm)

    return k(n_enc, e)


def _k2_body(ef_ref, gs_ref, gd_ref, *refs):
    o_ref = refs[-1]
    enc = refs[:8]
    wpe, wps, wpd, bp = refs[8], refs[9], refs[10], refs[11]
    proc_rest = refs[12:18]
    e_enc = _mlp_refs(ef_ref[...], enc)
    pre = (_f32dot(e_enc, wpe[...]) + _f32dot(gs_ref[0], wps[...])
           + _f32dot(gd_ref[0], wpd[...]) + bp[...])
    h = jnp.maximum(pre, 0.0)
    o_ref[...] = _res_out(h, proc_rest)


def _tc_edge_mlp(ef, g, enc_ws, proc_ws):
    full = lambda a: pl.BlockSpec(a.shape, lambda i: (0,) * a.ndim)
    ws = list(enc_ws) + list(proc_ws)
    return pl.pallas_call(
        _k2_body,
        grid=(E // EBLK,),
        in_specs=[
            pl.BlockSpec((EBLK, D_EDGE), lambda i: (i, 0)),
            pl.BlockSpec((1, EBLK, W), lambda i: (0, i, 0)),
            pl.BlockSpec((1, EBLK, W), lambda i: (1, i, 0)),
        ] + [full(a) for a in ws],
        out_specs=pl.BlockSpec((EBLK, W), lambda i: (i, 0)),
        out_shape=jax.ShapeDtypeStruct((E, W), jnp.float32),
    )(ef, g, g, *ws)


def _sc_scatter(e_proc, e):
    mesh = plsc.VectorSubcoreMesh(core_axis_name="c", subcore_axis_name="s")

    @functools.partial(
        pl.kernel,
        out_type=jax.ShapeDtypeStruct((NCORE, N, W), jnp.float32),
        mesh=mesh,
        compiler_params=pltpu.CompilerParams(use_tc_tiling_on_sc=False),
        scratch_types=[
            pltpu.VMEM_SHARED((N, W), jnp.float32),
            pltpu.VMEM((RPT, W), jnp.float32),
        ],
    )
    def k(x_hbm, idx_hbm, out_hbm, acc, zbuf):
        cid = lax.axis_index("c")
        sid = lax.axis_index("s")

        @pl.loop(0, RPT)
        def _(r):
            for c in range(0, W, 16):
                zbuf[r, pl.ds(c, 16)] = jnp.zeros((16,), jnp.float32)

        pltpu.sync_copy(zbuf, acc.at[pl.ds(sid * RPT, RPT)])
        plsc.subcore_barrier()

        def body(x_vmem, i_vmem):
            pltpu.sync_copy(x_vmem, acc.at[i_vmem.at[0]], add=True)

        pltpu.emit_pipeline(
            body,
            grid=(E // GW,),
            in_specs=[pl.BlockSpec((GW, W), lambda i: (i, 0)),
                      pl.BlockSpec((1, GW), lambda i: (0, i))],
            out_specs=[],
            core_axis_name=("c", "s"),
            dimension_semantics=(pltpu.PARALLEL,),
        )(x_hbm, idx_hbm)

        plsc.subcore_barrier()
        pltpu.sync_copy(acc.at[pl.ds(sid * RPT, RPT)],
                        out_hbm.at[cid, pl.ds(sid * RPT, RPT)])

    return k(e_proc, e)


def _k3_body(nf_ref, ne_ref, agg_ref, *refs):
    o_ref = refs[-1]
    wna, wnb, bn = refs[0], refs[1], refs[2]
    nproc_rest = refs[3:9]
    dec = refs[9:17]
    a = agg_ref[0] + agg_ref[1]
    pre = _f32dot(ne_ref[...], wna[...]) + _f32dot(a, wnb[...]) + bn[...]
    n_proc = _res_out(jnp.maximum(pre, 0.0), nproc_rest)
    o_ref[...] = nf_ref[...] + _mlp_refs(n_proc, dec)


def _tc_node_post(nf, n_enc, agg, nproc_ws, dec_ws):
    ws = list(nproc_ws) + list(dec_ws)
    return pl.pallas_call(
        _k3_body,
        out_shape=jax.ShapeDtypeStruct((N, D_NODE), jnp.float32),
    )(nf, n_enc, agg, *ws)


def kernel(edges, node_features, edge_features, params):
    e = edges[0]
    nf = node_features[0]
    ef = edge_features[0]

    enc_n = _mlp_params(params["node_encoder"])
    enc_e = _mlp_params(params["edge_encoder"])
    pp = params["edge_processor"]
    wp_in = pp["in"]["W"]
    proc_ws = [wp_in[:W], wp_in[W:2 * W], wp_in[2 * W:], pp["in"]["b"].reshape(1, -1),
               pp["blocks"][0]["a"]["W"], pp["blocks"][0]["a"]["b"].reshape(1, -1),
               pp["blocks"][0]["b"]["W"], pp["blocks"][0]["b"]["b"].reshape(1, -1),
               pp["out"]["W"], pp["out"]["b"].reshape(1, -1)]
    np_ = params["node_processor"]
    wn_in = np_["in"]["W"]
    nproc_ws = [wn_in[:W], wn_in[W:], np_["in"]["b"].reshape(1, -1),
                np_["blocks"][0]["a"]["W"], np_["blocks"][0]["a"]["b"].reshape(1, -1),
                np_["blocks"][0]["b"]["W"], np_["blocks"][0]["b"]["b"].reshape(1, -1),
                np_["out"]["W"], np_["out"]["b"].reshape(1, -1)]
    dec_ws = _mlp_params(params["decoder"])

    n_enc = _tc_node_encoder(nf, enc_n)
    g = _sc_gather(n_enc, e)
    e_proc = _tc_edge_mlp(ef, g, enc_e, proc_ws)
    agg = _sc_scatter(e_proc, e)
    out = _tc_node_post(nf, n_enc, agg, nproc_ws, dec_ws)
    return out[None]


# speedup vs baseline: 7.9244x; 1.0354x over previous
import functools

import jax
import jax.numpy as jnp
from jax import lax
from jax.experimental import pallas as pl
from jax.experimental.pallas import tpu as pltpu
from jax.experimental.pallas import tpu_sc as plsc

N = 10000
E = 320000
D_NODE = 128
D_EDGE = 16
W = 64

GW = 128
EBLK = 4000
NSUB = 16
NCORE = 2
RPT = N // NSUB


def _f32dot(x, w):
    return jnp.dot(x, w, preferred_element_type=jnp.float32)


def _bdot(x, w):
    return jnp.dot(x.astype(jnp.bfloat16), w, preferred_element_type=jnp.float32)


def _bmlp_refs(x, r):
    h = jnp.maximum(_bdot(x, r[0][...]) + r[1][...], 0.0)
    h1 = jnp.maximum(_bdot(h, r[2][...]) + r[3][...], 0.0)
    h2 = jnp.maximum(_bdot(h1, r[4][...]) + r[5][...], 0.0)
    h = h + h2
    return _bdot(h, r[6][...]) + r[7][...]


def _bres_out(h, r):
    h1 = jnp.maximum(_bdot(h, r[0][...]) + r[1][...], 0.0)
    h2 = jnp.maximum(_bdot(h1, r[2][...]) + r[3][...], 0.0)
    h = h + h2
    return _bdot(h, r[4][...]) + r[5][...]


def _mlp_refs(x, r):
    h = jnp.maximum(_f32dot(x, r[0][...]) + r[1][...], 0.0)
    h1 = jnp.maximum(_f32dot(h, r[2][...]) + r[3][...], 0.0)
    h2 = jnp.maximum(_f32dot(h1, r[4][...]) + r[5][...], 0.0)
    h = h + h2
    return _f32dot(h, r[6][...]) + r[7][...]


def _res_out(h, r):
    h1 = jnp.maximum(_f32dot(h, r[0][...]) + r[1][...], 0.0)
    h2 = jnp.maximum(_f32dot(h1, r[2][...]) + r[3][...], 0.0)
    h = h + h2
    return _f32dot(h, r[4][...]) + r[5][...]


def _mlp_params(p):
    return [p["in"]["W"], p["in"]["b"].reshape(1, -1),
            p["blocks"][0]["a"]["W"], p["blocks"][0]["a"]["b"].reshape(1, -1),
            p["blocks"][0]["b"]["W"], p["blocks"][0]["b"]["b"].reshape(1, -1),
            p["out"]["W"], p["out"]["b"].reshape(1, -1)]


def _k1_body(x_ref, *refs):
    o_ref = refs[-1]
    o_ref[...] = _mlp_refs(x_ref[...], refs[:-1])


def _tc_node_encoder(nf, ws):
    return pl.pallas_call(
        _k1_body,
        out_shape=jax.ShapeDtypeStruct((N, W), jnp.float32),
    )(nf, *ws)


def _sc_gather(n_enc, e):
    mesh = plsc.VectorSubcoreMesh(core_axis_name="c", subcore_axis_name="s")

    @functools.partial(
        pl.kernel,
        out_type=jax.ShapeDtypeStruct((2, E, W), jnp.float32),
        mesh=mesh,
        compiler_params=pltpu.CompilerParams(use_tc_tiling_on_sc=False),
    )
    def k(tab_hbm, idx_hbm, out_hbm):
        def body(i_vmem, o_vmem):
            pltpu.sync_copy(tab_hbm.at[i_vmem.at[0]], o_vmem.at[0])

        pltpu.emit_pipeline(
            body,
            grid=(2, E // GW),
            in_specs=[pl.BlockSpec((1, GW), lambda j, i: (j, i))],
            out_specs=[pl.BlockSpec((1, GW, W), lambda j, i: (j, i, 0))],
            core_axis_name=("c", "s"),
            dimension_semantics=(pltpu.PARALLEL, pltpu.PARALLEL),
        )(idx_hbm, out_hbm)

    return k(n_enc, e)


def _k2_body(ef_ref, gs_ref, gd_ref, *refs):
    o_ref = refs[-1]
    enc = refs[:8]
    wpe, wps, wpd, bp = refs[8], refs[9], refs[10], refs[11]
    proc_rest = refs[12:18]
    e_enc = _bmlp_refs(ef_ref[...], enc)
    pre = (_bdot(e_enc, wpe[...]) + _bdot(gs_ref[0], wps[...])
           + _bdot(gd_ref[0], wpd[...]) + bp[...])
    h = jnp.maximum(pre, 0.0)
    o_ref[...] = _bres_out(h, proc_rest)


def _tc_edge_mlp(ef, g, enc_ws, proc_ws):
    full = lambda a: pl.BlockSpec(a.shape, lambda i: (0,) * a.ndim)
    ws = [a.astype(jnp.bfloat16) if a.shape[0] > 1 else a
          for a in list(enc_ws) + list(proc_ws)]
    return pl.pallas_call(
        _k2_body,
        grid=(E // EBLK,),
        in_specs=[
            pl.BlockSpec((EBLK, D_EDGE), lambda i: (i, 0)),
            pl.BlockSpec((1, EBLK, W), lambda i: (0, i, 0)),
            pl.BlockSpec((1, EBLK, W), lambda i: (1, i, 0)),
        ] + [full(a) for a in ws],
        out_specs=pl.BlockSpec((EBLK, W), lambda i: (i, 0)),
        out_shape=jax.ShapeDtypeStruct((E, W), jnp.float32),
    )(ef, g, g, *ws)


def _sc_scatter(e_proc, e):
    mesh = plsc.VectorSubcoreMesh(core_axis_name="c", subcore_axis_name="s")

    @functools.partial(
        pl.kernel,
        out_type=jax.ShapeDtypeStruct((NCORE, N, W), jnp.float32),
        mesh=mesh,
        compiler_params=pltpu.CompilerParams(use_tc_tiling_on_sc=False),
        scratch_types=[
            pltpu.VMEM_SHARED((N, W), jnp.float32),
            pltpu.VMEM((RPT, W), jnp.float32),
        ],
    )
    def k(x_hbm, idx_hbm, out_hbm, acc, zbuf):
        cid = lax.axis_index("c")
        sid = lax.axis_index("s")

        @pl.loop(0, RPT)
        def _(r):
            for c in range(0, W, 16):
                zbuf[r, pl.ds(c, 16)] = jnp.zeros((16,), jnp.float32)

        pltpu.sync_copy(zbuf, acc.at[pl.ds(sid * RPT, RPT)])
        plsc.subcore_barrier()

        def body(x_vmem, i_vmem):
            pltpu.sync_copy(x_vmem, acc.at[i_vmem.at[0]], add=True)

        pltpu.emit_pipeline(
            body,
            grid=(E // GW,),
            in_specs=[pl.BlockSpec((GW, W), lambda i: (i, 0)),
                      pl.BlockSpec((1, GW), lambda i: (0, i))],
            out_specs=[],
            core_axis_name=("c", "s"),
            dimension_semantics=(pltpu.PARALLEL,),
        )(x_hbm, idx_hbm)

        plsc.subcore_barrier()
        pltpu.sync_copy(acc.at[pl.ds(sid * RPT, RPT)],
                        out_hbm.at[cid, pl.ds(sid * RPT, RPT)])

    return k(e_proc, e)


def _k3_body(nf_ref, ne_ref, agg_ref, *refs):
    o_ref = refs[-1]
    wna, wnb, bn = refs[0], refs[1], refs[2]
    nproc_rest = refs[3:9]
    dec = refs[9:17]
    a = agg_ref[0] + agg_ref[1]
    pre = _f32dot(ne_ref[...], wna[...]) + _f32dot(a, wnb[...]) + bn[...]
    n_proc = _res_out(jnp.maximum(pre, 0.0), nproc_rest)
    o_ref[...] = nf_ref[...] + _mlp_refs(n_proc, dec)


def _tc_node_post(nf, n_enc, agg, nproc_ws, dec_ws):
    ws = list(nproc_ws) + list(dec_ws)
    return pl.pallas_call(
        _k3_body,
        out_shape=jax.ShapeDtypeStruct((N, D_NODE), jnp.float32),
    )(nf, n_enc, agg, *ws)


def kernel(edges, node_features, edge_features, params):
    e = edges[0]
    nf = node_features[0]
    ef = edge_features[0]

    enc_n = _mlp_params(params["node_encoder"])
    enc_e = _mlp_params(params["edge_encoder"])
    pp = params["edge_processor"]
    wp_in = pp["in"]["W"]
    proc_ws = [wp_in[:W], wp_in[W:2 * W], wp_in[2 * W:], pp["in"]["b"].reshape(1, -1),
               pp["blocks"][0]["a"]["W"], pp["blocks"][0]["a"]["b"].reshape(1, -1),
               pp["blocks"][0]["b"]["W"], pp["blocks"][0]["b"]["b"].reshape(1, -1),
               pp["out"]["W"], pp["out"]["b"].reshape(1, -1)]
    np_ = params["node_processor"]
    wn_in = np_["in"]["W"]
    nproc_ws = [wn_in[:W], wn_in[W:], np_["in"]["b"].reshape(1, -1),
                np_["blocks"][0]["a"]["W"], np_["blocks"][0]["a"]["b"].reshape(1, -1),
                np_["blocks"][0]["b"]["W"], np_["blocks"][0]["b"]["b"].reshape(1, -1),
                np_["out"]["W"], np_["out"]["b"].reshape(1, -1)]
    dec_ws = _mlp_params(params["decoder"])

    n_enc = _tc_node_encoder(nf, enc_n)
    g = _sc_gather(n_enc, e)
    e_proc = _tc_edge_mlp(ef, g, enc_e, proc_ws)
    agg = _sc_scatter(e_proc, e)
    out = _tc_node_post(nf, n_enc, agg, nproc_ws, dec_ws)
    return out[None]


# speedup vs baseline: 12.2588x; 1.5470x over previous
import functools

import jax
import jax.numpy as jnp
from jax import lax
from jax.experimental import pallas as pl
from jax.experimental.pallas import tpu as pltpu
from jax.experimental.pallas import tpu_sc as plsc

N = 10000
E = 320000
D_NODE = 128
D_EDGE = 16
W = 64

GW = 128
WPB = 2
PBLK = 2000
NSUB = 16
NCORE = 2
RPT = N // NSUB


def _f32dot(x, w):
    return jnp.dot(x, w, preferred_element_type=jnp.float32)


def _bdot(x, w):
    return jnp.dot(x.astype(jnp.bfloat16), w, preferred_element_type=jnp.float32)


def _mlp_refs(x, r):
    h = jnp.maximum(_f32dot(x, r[0][...]) + r[1][...], 0.0)
    h1 = jnp.maximum(_f32dot(h, r[2][...]) + r[3][...], 0.0)
    h2 = jnp.maximum(_f32dot(h1, r[4][...]) + r[5][...], 0.0)
    h = h + h2
    return _f32dot(h, r[6][...]) + r[7][...]


def _res_out(h, r):
    h1 = jnp.maximum(_f32dot(h, r[0][...]) + r[1][...], 0.0)
    h2 = jnp.maximum(_f32dot(h1, r[2][...]) + r[3][...], 0.0)
    h = h + h2
    return _f32dot(h, r[4][...]) + r[5][...]


def _mlp_params(p):
    return [p["in"]["W"], p["in"]["b"].reshape(1, -1),
            p["blocks"][0]["a"]["W"], p["blocks"][0]["a"]["b"].reshape(1, -1),
            p["blocks"][0]["b"]["W"], p["blocks"][0]["b"]["b"].reshape(1, -1),
            p["out"]["W"], p["out"]["b"].reshape(1, -1)]


def _bdiag(w):
    z = jnp.zeros(w.shape, w.dtype)
    return jnp.concatenate(
        [jnp.concatenate([w, z], axis=1), jnp.concatenate([z, w], axis=1)],
        axis=0).astype(jnp.bfloat16)


def _bpair(b):
    b = b.reshape(1, -1)
    return jnp.concatenate([b, b], axis=1)


def _k1_body(x_ref, wps_ref, wpd_ref, *refs):
    ne_ref, us_ref, ud_ref = refs[-3], refs[-2], refs[-1]
    n_enc = _mlp_refs(x_ref[...], refs[:-3])
    ne_ref[...] = n_enc
    us_ref[...] = _f32dot(n_enc, wps_ref[...])
    ud_ref[...] = _f32dot(n_enc, wpd_ref[...])


def _tc_node_encoder(nf, ws, wps, wpd):
    return pl.pallas_call(
        _k1_body,
        out_shape=[jax.ShapeDtypeStruct((N, W), jnp.float32),
                   jax.ShapeDtypeStruct((N, W), jnp.float32),
                   jax.ShapeDtypeStruct((N, W), jnp.float32)],
    )(nf, wps, wpd, *ws)


def _sc_gather(u_s, u_d, e):
    mesh = plsc.VectorSubcoreMesh(core_axis_name="c", subcore_axis_name="s")
    eb = GW * WPB

    @functools.partial(
        pl.kernel,
        out_type=jax.ShapeDtypeStruct((E // 2, 2 * W), jnp.float32),
        mesh=mesh,
        compiler_params=pltpu.CompilerParams(use_tc_tiling_on_sc=False),
        scratch_types=[
            pltpu.VMEM((eb, W), jnp.float32),
            pltpu.VMEM((eb, W), jnp.float32),
            pltpu.SemaphoreType.DMA,
        ],
    )
    def k(us_hbm, ud_hbm, idx_hbm, out_hbm, bs, bd, sem):
        def body(i_vmem, o_vmem):
            cps = []
            for wnd in range(WPB):
                sl = pl.ds(wnd * GW, GW)
                cps.append(pltpu.async_copy(
                    us_hbm.at[i_vmem.at[0, sl]], bs.at[sl], sem))
                cps.append(pltpu.async_copy(
                    ud_hbm.at[i_vmem.at[1, sl]], bd.at[sl], sem))
            for cp in cps:
                cp.wait()

            @pl.loop(0, eb // 2)
            def _(r):
                for c in range(0, W, 16):
                    o_vmem[r, pl.ds(c, 16)] = (
                        bs[2 * r, pl.ds(c, 16)] + bd[2 * r, pl.ds(c, 16)])
                    o_vmem[r, pl.ds(W + c, 16)] = (
                        bs[2 * r + 1, pl.ds(c, 16)] + bd[2 * r + 1, pl.ds(c, 16)])

        pltpu.emit_pipeline(
            body,
            grid=(E // eb,),
            in_specs=[pl.BlockSpec((2, eb), lambda i: (0, i))],
            out_specs=[pl.BlockSpec((eb // 2, 2 * W), lambda i: (i, 0))],
            core_axis_name=("c", "s"),
            dimension_semantics=(pltpu.PARALLEL,),
        )(idx_hbm, out_hbm)

    return k(u_s, u_d, e)


def _k2_body(ef_ref, g_ref, *refs):
    o_ref = refs[-1]
    w1, b1, wa, ba, wb, bb, wo, bo = refs[0:8]
    wpe, bp, wpa, bpa, wpb, bpb, wpo, bpo = refs[8:16]
    h = jnp.maximum(_bdot(ef_ref[...], w1[...]) + b1[...], 0.0)
    h1 = jnp.maximum(_bdot(h, wa[...]) + ba[...], 0.0)
    h2 = jnp.maximum(_bdot(h1, wb[...]) + bb[...], 0.0)
    h = h + h2
    e_enc = _bdot(h, wo[...]) + bo[...]
    pre = _bdot(e_enc, wpe[...]) + g_ref[...] + bp[...]
    h = jnp.maximum(pre, 0.0)
    h1 = jnp.maximum(_bdot(h, wpa[...]) + bpa[...], 0.0)
    h2 = jnp.maximum(_bdot(h1, wpb[...]) + bpb[...], 0.0)
    h = h + h2
    o_ref[...] = _bdot(h, wpo[...]) + bpo[...]


def _tc_edge_mlp(ef2, g2, ws):
    full = lambda a: pl.BlockSpec(a.shape, lambda i: (0,) * a.ndim)
    return pl.pallas_call(
        _k2_body,
        grid=(E // 2 // PBLK,),
        in_specs=[
            pl.BlockSpec((PBLK, 2 * D_EDGE), lambda i: (i, 0)),
            pl.BlockSpec((PBLK, 2 * W), lambda i: (i, 0)),
        ] + [full(a) for a in ws],
        out_specs=pl.BlockSpec((PBLK, 2 * W), lambda i: (i, 0)),
        out_shape=jax.ShapeDtypeStruct((E // 2, 2 * W), jnp.float32),
    )(ef2, g2, *ws)


def _sc_scatter(e_proc, e):
    mesh = plsc.VectorSubcoreMesh(core_axis_name="c", subcore_axis_name="s")

    @functools.partial(
        pl.kernel,
        out_type=jax.ShapeDtypeStruct((NCORE, N, W), jnp.float32),
        mesh=mesh,
        compiler_params=pltpu.CompilerParams(use_tc_tiling_on_sc=False),
        scratch_types=[
            pltpu.VMEM_SHARED((N, W), jnp.float32),
            pltpu.VMEM((RPT, W), jnp.float32),
        ],
    )
    def k(x_hbm, idx_hbm, out_hbm, acc, zbuf):
        cid = lax.axis_index("c")
        sid = lax.axis_index("s")

        @pl.loop(0, RPT)
        def _(r):
            for c in range(0, W, 16):
                zbuf[r, pl.ds(c, 16)] = jnp.zeros((16,), jnp.float32)

        pltpu.sync_copy(zbuf, acc.at[pl.ds(sid * RPT, RPT)])
        plsc.subcore_barrier()

        def body(x_vmem, i_vmem):
            pltpu.sync_copy(x_vmem, acc.at[i_vmem.at[0]], add=True)

        pltpu.emit_pipeline(
            body,
            grid=(E // GW,),
            in_specs=[pl.BlockSpec((GW, W), lambda i: (i, 0)),
                      pl.BlockSpec((1, GW), lambda i: (0, i))],
            out_specs=[],
            core_axis_name=("c", "s"),
            dimension_semantics=(pltpu.PARALLEL,),
        )(x_hbm, idx_hbm)

        plsc.subcore_barrier()
        pltpu.sync_copy(acc.at[pl.ds(sid * RPT, RPT)],
                        out_hbm.at[cid, pl.ds(sid * RPT, RPT)])

    return k(e_proc, e)


def _k3_body(nf_ref, ne_ref, agg_ref, *refs):
    o_ref = refs[-1]
    wna, wnb, bn = refs[0], refs[1], refs[2]
    nproc_rest = refs[3:9]
    dec = refs[9:17]
    a = agg_ref[0] + agg_ref[1]
    pre = _f32dot(ne_ref[...], wna[...]) + _f32dot(a, wnb[...]) + bn[...]
    n_proc = _res_out(jnp.maximum(pre, 0.0), nproc_rest)
    o_ref[...] = nf_ref[...] + _mlp_refs(n_proc, dec)


def _tc_node_post(nf, n_enc, agg, nproc_ws, dec_ws):
    ws = list(nproc_ws) + list(dec_ws)
    return pl.pallas_call(
        _k3_body,
        out_shape=jax.ShapeDtypeStruct((N, D_NODE), jnp.float32),
    )(nf, n_enc, agg, *ws)


def kernel(edges, node_features, edge_features, params):
    e = edges[0]
    nf = node_features[0]
    ef = edge_features[0]

    enc_n = _mlp_params(params["node_encoder"])
    ee = params["edge_encoder"]
    pp = params["edge_processor"]
    wp_in = pp["in"]["W"]

    edge_ws = [
        _bdiag(ee["in"]["W"]), _bpair(ee["in"]["b"]),
        _bdiag(ee["blocks"][0]["a"]["W"]), _bpair(ee["blocks"][0]["a"]["b"]),
        _bdiag(ee["blocks"][0]["b"]["W"]), _bpair(ee["blocks"][0]["b"]["b"]),
        _bdiag(ee["out"]["W"]), _bpair(ee["out"]["b"]),
        _bdiag(wp_in[:W]), _bpair(pp["in"]["b"]),
        _bdiag(pp["blocks"][0]["a"]["W"]), _bpair(pp["blocks"][0]["a"]["b"]),
        _bdiag(pp["blocks"][0]["b"]["W"]), _bpair(pp["blocks"][0]["b"]["b"]),
        _bdiag(pp["out"]["W"]), _bpair(pp["out"]["b"]),
    ]

    np_ = params["node_processor"]
    wn_in = np_["in"]["W"]
    nproc_ws = [wn_in[:W], wn_in[W:], np_["in"]["b"].reshape(1, -1),
                np_["blocks"][0]["a"]["W"], np_["blocks"][0]["a"]["b"].reshape(1, -1),
                np_["blocks"][0]["b"]["W"], np_["blocks"][0]["b"]["b"].reshape(1, -1),
                np_["out"]["W"], np_["out"]["b"].reshape(1, -1)]
    dec_ws = _mlp_params(params["decoder"])

    n_enc, u_s, u_d = _tc_node_encoder(nf, enc_n, wp_in[W:2 * W], wp_in[2 * W:])
    g2 = _sc_gather(u_s, u_d, e)
    ef2 = ef.reshape(E // 2, 2 * D_EDGE)
    e_proc2 = _tc_edge_mlp(ef2, g2, edge_ws)
    agg = _sc_scatter(e_proc2.reshape(E, W), e)
    out = _tc_node_post(nf, n_enc, agg, nproc_ws, dec_ws)
    return out[None]


# speedup vs baseline: 14.6474x; 1.1948x over previous
import functools

import jax
import jax.numpy as jnp
from jax import lax
from jax.experimental import pallas as pl
from jax.experimental.pallas import tpu as pltpu
from jax.experimental.pallas import tpu_sc as plsc

N = 10000
E = 320000
D_NODE = 128
D_EDGE = 16
W = 64

GW = 128
WPB = 2
PBLK = 2000
NSUB = 16
NCORE = 2
RPT = N // NSUB


def _f32dot(x, w):
    return jnp.dot(x, w, preferred_element_type=jnp.float32)


def _bdot(x, w):
    return jnp.dot(x.astype(jnp.bfloat16), w, preferred_element_type=jnp.float32)


def _mlp_refs(x, r):
    h = jnp.maximum(_f32dot(x, r[0][...]) + r[1][...], 0.0)
    h1 = jnp.maximum(_f32dot(h, r[2][...]) + r[3][...], 0.0)
    h2 = jnp.maximum(_f32dot(h1, r[4][...]) + r[5][...], 0.0)
    h = h + h2
    return _f32dot(h, r[6][...]) + r[7][...]


def _res_out(h, r):
    h1 = jnp.maximum(_f32dot(h, r[0][...]) + r[1][...], 0.0)
    h2 = jnp.maximum(_f32dot(h1, r[2][...]) + r[3][...], 0.0)
    h = h + h2
    return _f32dot(h, r[4][...]) + r[5][...]


def _mlp_params(p):
    return [p["in"]["W"], p["in"]["b"].reshape(1, -1),
            p["blocks"][0]["a"]["W"], p["blocks"][0]["a"]["b"].reshape(1, -1),
            p["blocks"][0]["b"]["W"], p["blocks"][0]["b"]["b"].reshape(1, -1),
            p["out"]["W"], p["out"]["b"].reshape(1, -1)]


def _bdiag(w):
    z = jnp.zeros(w.shape, w.dtype)
    return jnp.concatenate(
        [jnp.concatenate([w, z], axis=1), jnp.concatenate([z, w], axis=1)],
        axis=0).astype(jnp.bfloat16)


def _bpair(b):
    b = b.reshape(1, -1)
    return jnp.concatenate([b, b], axis=1)


def _k1_body(x_ref, wps_ref, wpd_ref, *refs):
    ne_ref, us_ref, ud_ref = refs[-3], refs[-2], refs[-1]
    n_enc = _mlp_refs(x_ref[...], refs[:-3])
    ne_ref[...] = n_enc
    us_ref[...] = _f32dot(n_enc, wps_ref[...])
    ud_ref[...] = _f32dot(n_enc, wpd_ref[...])


def _tc_node_encoder(nf, ws, wps, wpd):
    return pl.pallas_call(
        _k1_body,
        out_shape=[jax.ShapeDtypeStruct((N, W), jnp.float32),
                   jax.ShapeDtypeStruct((N, W), jnp.float32),
                   jax.ShapeDtypeStruct((N, W), jnp.float32)],
    )(nf, wps, wpd, *ws)


def _sc_gather(u_s, u_d, e):
    mesh = plsc.VectorSubcoreMesh(core_axis_name="c", subcore_axis_name="s")
    eb = GW * WPB

    @functools.partial(
        pl.kernel,
        out_type=[jax.ShapeDtypeStruct((E, W), jnp.float32),
                  jax.ShapeDtypeStruct((E, W), jnp.float32)],
        mesh=mesh,
        compiler_params=pltpu.CompilerParams(use_tc_tiling_on_sc=False),
        scratch_types=[pltpu.SemaphoreType.DMA],
    )
    def k(us_hbm, ud_hbm, idx_hbm, os_hbm, od_hbm, sem):
        def body(i_vmem, os_vmem, od_vmem):
            cps = []
            for wnd in range(WPB):
                sl = pl.ds(wnd * GW, GW)
                cps.append(pltpu.async_copy(
                    us_hbm.at[i_vmem.at[0, sl]], os_vmem.at[sl], sem))
                cps.append(pltpu.async_copy(
                    ud_hbm.at[i_vmem.at[1, sl]], od_vmem.at[sl], sem))
            for cp in cps:
                cp.wait()

        pltpu.emit_pipeline(
            body,
            grid=(E // eb,),
            in_specs=[pl.BlockSpec((2, eb), lambda i: (0, i))],
            out_specs=[pl.BlockSpec((eb, W), lambda i: (i, 0)),
                       pl.BlockSpec((eb, W), lambda i: (i, 0))],
            core_axis_name=("c", "s"),
            dimension_semantics=(pltpu.PARALLEL,),
        )(idx_hbm, os_hbm, od_hbm)

    return k(u_s, u_d, e)


def _k2_body(ef_ref, gs_ref, gd_ref, *refs):
    o_ref = refs[-1]
    w1, b1, wa, ba, wb, bb, wo, bo = refs[0:8]
    wpe, bp, wpa, bpa, wpb, bpb, wpo, bpo = refs[8:16]
    h = jnp.maximum(_bdot(ef_ref[...], w1[...]) + b1[...], 0.0)
    h1 = jnp.maximum(_bdot(h, wa[...]) + ba[...], 0.0)
    h2 = jnp.maximum(_bdot(h1, wb[...]) + bb[...], 0.0)
    h = h + h2
    e_enc = _bdot(h, wo[...]) + bo[...]
    pre = _bdot(e_enc, wpe[...]) + gs_ref[...] + gd_ref[...] + bp[...]
    h = jnp.maximum(pre, 0.0)
    h1 = jnp.maximum(_bdot(h, wpa[...]) + bpa[...], 0.0)
    h2 = jnp.maximum(_bdot(h1, wpb[...]) + bpb[...], 0.0)
    h = h + h2
    o_ref[...] = _bdot(h, wpo[...]) + bpo[...]


def _tc_edge_mlp(ef2, gs2, gd2, ws):
    full = lambda a: pl.BlockSpec(a.shape, lambda i: (0,) * a.ndim)
    return pl.pallas_call(
        _k2_body,
        grid=(E // 2 // PBLK,),
        in_specs=[
            pl.BlockSpec((PBLK, 2 * D_EDGE), lambda i: (i, 0)),
            pl.BlockSpec((PBLK, 2 * W), lambda i: (i, 0)),
            pl.BlockSpec((PBLK, 2 * W), lambda i: (i, 0)),
        ] + [full(a) for a in ws],
        out_specs=pl.BlockSpec((PBLK, 2 * W), lambda i: (i, 0)),
        out_shape=jax.ShapeDtypeStruct((E // 2, 2 * W), jnp.float32),
    )(ef2, gs2, gd2, *ws)


def _sc_scatter(e_proc, e):
    mesh = plsc.VectorSubcoreMesh(core_axis_name="c", subcore_axis_name="s")

    @functools.partial(
        pl.kernel,
        out_type=jax.ShapeDtypeStruct((NCORE, N, W), jnp.float32),
        mesh=mesh,
        compiler_params=pltpu.CompilerParams(use_tc_tiling_on_sc=False),
        scratch_types=[
            pltpu.VMEM_SHARED((N, W), jnp.float32),
            pltpu.VMEM((RPT, W), jnp.float32),
        ],
    )
    def k(x_hbm, idx_hbm, out_hbm, acc, zbuf):
        cid = lax.axis_index("c")
        sid = lax.axis_index("s")

        @pl.loop(0, RPT)
        def _(r):
            for c in range(0, W, 16):
                zbuf[r, pl.ds(c, 16)] = jnp.zeros((16,), jnp.float32)

        pltpu.sync_copy(zbuf, acc.at[pl.ds(sid * RPT, RPT)])
        plsc.subcore_barrier()

        def body(x_vmem, i_vmem):
            pltpu.sync_copy(x_vmem, acc.at[i_vmem.at[0]], add=True)

        pltpu.emit_pipeline(
            body,
            grid=(E // GW,),
            in_specs=[pl.BlockSpec((GW, W), lambda i: (i, 0)),
                      pl.BlockSpec((1, GW), lambda i: (0, i))],
            out_specs=[],
            core_axis_name=("c", "s"),
            dimension_semantics=(pltpu.PARALLEL,),
        )(x_hbm, idx_hbm)

        plsc.subcore_barrier()
        pltpu.sync_copy(acc.at[pl.ds(sid * RPT, RPT)],
                        out_hbm.at[cid, pl.ds(sid * RPT, RPT)])

    return k(e_proc, e)


def _k3_body(nf_ref, ne_ref, agg_ref, *refs):
    o_ref = refs[-1]
    wna, wnb, bn = refs[0], refs[1], refs[2]
    nproc_rest = refs[3:9]
    dec = refs[9:17]
    a = agg_ref[0] + agg_ref[1]
    pre = _f32dot(ne_ref[...], wna[...]) + _f32dot(a, wnb[...]) + bn[...]
    n_proc = _res_out(jnp.maximum(pre, 0.0), nproc_rest)
    o_ref[...] = nf_ref[...] + _mlp_refs(n_proc, dec)


def _tc_node_post(nf, n_enc, agg, nproc_ws, dec_ws):
    ws = list(nproc_ws) + list(dec_ws)
    return pl.pallas_call(
        _k3_body,
        out_shape=jax.ShapeDtypeStruct((N, D_NODE), jnp.float32),
    )(nf, n_enc, agg, *ws)


def kernel(edges, node_features, edge_features, params):
    e = edges[0]
    nf = node_features[0]
    ef = edge_features[0]

    enc_n = _mlp_params(params["node_encoder"])
    ee = params["edge_encoder"]
    pp = params["edge_processor"]
    wp_in = pp["in"]["W"]

    edge_ws = [
        _bdiag(ee["in"]["W"]), _bpair(ee["in"]["b"]),
        _bdiag(ee["blocks"][0]["a"]["W"]), _bpair(ee["blocks"][0]["a"]["b"]),
        _bdiag(ee["blocks"][0]["b"]["W"]), _bpair(ee["blocks"][0]["b"]["b"]),
        _bdiag(ee["out"]["W"]), _bpair(ee["out"]["b"]),
        _bdiag(wp_in[:W]), _bpair(pp["in"]["b"]),
        _bdiag(pp["blocks"][0]["a"]["W"]), _bpair(pp["blocks"][0]["a"]["b"]),
        _bdiag(pp["blocks"][0]["b"]["W"]), _bpair(pp["blocks"][0]["b"]["b"]),
        _bdiag(pp["out"]["W"]), _bpair(pp["out"]["b"]),
    ]

    np_ = params["node_processor"]
    wn_in = np_["in"]["W"]
    nproc_ws = [wn_in[:W], wn_in[W:], np_["in"]["b"].reshape(1, -1),
                np_["blocks"][0]["a"]["W"], np_["blocks"][0]["a"]["b"].reshape(1, -1),
                np_["blocks"][0]["b"]["W"], np_["blocks"][0]["b"]["b"].reshape(1, -1),
                np_["out"]["W"], np_["out"]["b"].reshape(1, -1)]
    dec_ws = _mlp_params(params["decoder"])

    n_enc, u_s, u_d = _tc_node_encoder(nf, enc_n, wp_in[W:2 * W], wp_in[2 * W:])
    gs, gd = _sc_gather(u_s, u_d, e)
    ef2 = ef.reshape(E // 2, 2 * D_EDGE)
    e_proc2 = _tc_edge_mlp(ef2, gs.reshape(E // 2, 2 * W),
                           gd.reshape(E // 2, 2 * W), edge_ws)
    agg = _sc_scatter(e_proc2.reshape(E, W), e)
    out = _tc_node_post(nf, n_enc, agg, nproc_ws, dec_ws)
    return out[None]


# speedup vs baseline: 20.6266x; 1.4082x over previous
import functools

import jax
import jax.numpy as jnp
from jax import lax
from jax.experimental import pallas as pl
from jax.experimental.pallas import tpu as pltpu
from jax.experimental.pallas import tpu_sc as plsc

N = 10000
E = 320000
D_NODE = 128
D_EDGE = 16
W = 64

GW = 128
WPB = 2
PBLK = 3200
NSUB = 16
NCORE = 2
RPT = N // NSUB


def _f32dot(x, w):
    return jnp.dot(x, w, preferred_element_type=jnp.float32)


def _bdot(x, w):
    return jnp.dot(x.astype(jnp.bfloat16), w, preferred_element_type=jnp.float32)


def _mlp_refs(x, r):
    h = jnp.maximum(_f32dot(x, r[0][...]) + r[1][...], 0.0)
    h1 = jnp.maximum(_f32dot(h, r[2][...]) + r[3][...], 0.0)
    h2 = jnp.maximum(_f32dot(h1, r[4][...]) + r[5][...], 0.0)
    h = h + h2
    return _f32dot(h, r[6][...]) + r[7][...]


def _res_out(h, r):
    h1 = jnp.maximum(_f32dot(h, r[0][...]) + r[1][...], 0.0)
    h2 = jnp.maximum(_f32dot(h1, r[2][...]) + r[3][...], 0.0)
    h = h + h2
    return _f32dot(h, r[4][...]) + r[5][...]


def _mlp_params(p):
    return [p["in"]["W"], p["in"]["b"].reshape(1, -1),
            p["blocks"][0]["a"]["W"], p["blocks"][0]["a"]["b"].reshape(1, -1),
            p["blocks"][0]["b"]["W"], p["blocks"][0]["b"]["b"].reshape(1, -1),
            p["out"]["W"], p["out"]["b"].reshape(1, -1)]


def _bdiag(w):
    z = jnp.zeros(w.shape, w.dtype)
    return jnp.concatenate(
        [jnp.concatenate([w, z], axis=1), jnp.concatenate([z, w], axis=1)],
        axis=0).astype(jnp.bfloat16)


def _bpair(b):
    b = b.reshape(1, -1)
    return jnp.concatenate([b, b], axis=1)


def _k1_body(x_ref, wps_ref, wpd_ref, *refs):
    ne_ref, us_ref, ud_ref = refs[-3], refs[-2], refs[-1]
    n_enc = _mlp_refs(x_ref[...], refs[:-3])
    ne_ref[...] = n_enc
    us_ref[...] = _f32dot(n_enc, wps_ref[...])
    ud_ref[...] = _f32dot(n_enc, wpd_ref[...])


def _tc_node_encoder(nf, ws, wps, wpd):
    return pl.pallas_call(
        _k1_body,
        out_shape=[jax.ShapeDtypeStruct((N, W), jnp.float32),
                   jax.ShapeDtypeStruct((N, W), jnp.float32),
                   jax.ShapeDtypeStruct((N, W), jnp.float32)],
    )(nf, wps, wpd, *ws)


def _sc_gather(u_s, u_d, e):
    mesh = plsc.VectorSubcoreMesh(core_axis_name="c", subcore_axis_name="s")
    eb = GW * WPB
    hb = (E // 2) // eb

    @functools.partial(
        pl.kernel,
        out_type=[jax.ShapeDtypeStruct((E // 2, 2 * W), jnp.float32),
                  jax.ShapeDtypeStruct((E // 2, 2 * W), jnp.float32)],
        mesh=mesh,
        compiler_params=pltpu.CompilerParams(use_tc_tiling_on_sc=False),
        scratch_types=[pltpu.SemaphoreType.DMA],
    )
    def k(us_hbm, ud_hbm, idx_hbm, os_hbm, od_hbm, sem):
        def body(i_vmem, os_vmem, od_vmem):
            cps = []
            for wnd in range(WPB):
                sl = pl.ds(wnd * GW, GW)
                cps.append(pltpu.async_copy(
                    us_hbm.at[i_vmem.at[0, sl]], os_vmem.at[sl], sem))
                cps.append(pltpu.async_copy(
                    ud_hbm.at[i_vmem.at[1, sl]], od_vmem.at[sl], sem))
            for cp in cps:
                cp.wait()

        pltpu.emit_pipeline(
            body,
            grid=(E // eb,),
            in_specs=[pl.BlockSpec((2, eb), lambda i: (0, i))],
            out_specs=[pl.BlockSpec((eb, W), lambda i: (i % hb, i // hb)),
                       pl.BlockSpec((eb, W), lambda i: (i % hb, i // hb))],
            core_axis_name=("c", "s"),
            dimension_semantics=(pltpu.PARALLEL,),
        )(idx_hbm, os_hbm, od_hbm)

    return k(u_s, u_d, e)


def _tdot(x, w):
    return lax.dot_general(x.astype(jnp.bfloat16), w,
                           (((0,), (0,)), ((), ())),
                           preferred_element_type=jnp.float32)


def _k2_body(eflo_ref, efhi_ref, gs_ref, gd_ref, *refs):
    o_ref = refs[-1]
    w1, b1, wa, ba, wb, bb, wo, bo = refs[0:8]
    wpe, bp, wpa, bpa, wpb, bpb, wpo, bpo = refs[8:16]
    h1 = jnp.concatenate([_tdot(eflo_ref[...], w1[...]),
                          _tdot(efhi_ref[...], w1[...])], axis=1)
    h = jnp.maximum(h1 + b1[...], 0.0)
    h1 = jnp.maximum(_bdot(h, wa[...]) + ba[...], 0.0)
    h2 = jnp.maximum(_bdot(h1, wb[...]) + bb[...], 0.0)
    h = h + h2
    e_enc = _bdot(h, wo[...]) + bo[...]
    pre = _bdot(e_enc, wpe[...]) + gs_ref[...] + gd_ref[...] + bp[...]
    h = jnp.maximum(pre, 0.0)
    h1 = jnp.maximum(_bdot(h, wpa[...]) + bpa[...], 0.0)
    h2 = jnp.maximum(_bdot(h1, wpb[...]) + bpb[...], 0.0)
    h = h + h2
    o_ref[...] = _bdot(h, wpo[...]) + bpo[...]


def _tc_edge_mlp(eft, gs2, gd2, ws):
    full = lambda a: pl.BlockSpec(a.shape, lambda i: (0,) * a.ndim)
    nb = (E // 2) // PBLK
    return pl.pallas_call(
        _k2_body,
        grid=(nb,),
        in_specs=[
            pl.BlockSpec((D_EDGE, PBLK), lambda i: (0, i)),
            pl.BlockSpec((D_EDGE, PBLK), lambda i: (0, i + nb)),
            pl.BlockSpec((PBLK, 2 * W), lambda i: (i, 0)),
            pl.BlockSpec((PBLK, 2 * W), lambda i: (i, 0)),
        ] + [full(a) for a in ws],
        out_specs=pl.BlockSpec((PBLK, 2 * W), lambda i: (i, 0)),
        out_shape=jax.ShapeDtypeStruct((E // 2, 2 * W), jnp.float32),
    )(eft, eft, gs2, gd2, *ws)


def _sc_scatter(e_proc2, e):
    mesh = plsc.VectorSubcoreMesh(core_axis_name="c", subcore_axis_name="s")
    hw = (E // 2) // GW

    @functools.partial(
        pl.kernel,
        out_type=jax.ShapeDtypeStruct((NCORE, N, W), jnp.float32),
        mesh=mesh,
        compiler_params=pltpu.CompilerParams(use_tc_tiling_on_sc=False),
        scratch_types=[
            pltpu.VMEM_SHARED((N, W), jnp.float32),
            pltpu.VMEM((RPT, W), jnp.float32),
        ],
    )
    def k(x_hbm, idx_hbm, out_hbm, acc, zbuf):
        cid = lax.axis_index("c")
        sid = lax.axis_index("s")

        @pl.loop(0, RPT)
        def _(r):
            for c in range(0, W, 16):
                zbuf[r, pl.ds(c, 16)] = jnp.zeros((16,), jnp.float32)

        pltpu.sync_copy(zbuf, acc.at[pl.ds(sid * RPT, RPT)])
        plsc.subcore_barrier()

        def body(x_vmem, i_vmem):
            pltpu.sync_copy(x_vmem, acc.at[i_vmem.at[0]], add=True)

        pltpu.emit_pipeline(
            body,
            grid=(E // GW,),
            in_specs=[pl.BlockSpec((GW, W), lambda i: (i % hw, i // hw)),
                      pl.BlockSpec((1, GW), lambda i: (0, i))],
            out_specs=[],
            core_axis_name=("c", "s"),
            dimension_semantics=(pltpu.PARALLEL,),
        )(x_hbm, idx_hbm)

        plsc.subcore_barrier()
        pltpu.sync_copy(acc.at[pl.ds(sid * RPT, RPT)],
                        out_hbm.at[cid, pl.ds(sid * RPT, RPT)])

    return k(e_proc2, e)


def _k3_body(nf_ref, ne_ref, agg_ref, *refs):
    o_ref = refs[-1]
    wna, wnb, bn = refs[0], refs[1], refs[2]
    nproc_rest = refs[3:9]
    dec = refs[9:17]
    a = agg_ref[0] + agg_ref[1]
    pre = _f32dot(ne_ref[...], wna[...]) + _f32dot(a, wnb[...]) + bn[...]
    n_proc = _res_out(jnp.maximum(pre, 0.0), nproc_rest)
    o_ref[...] = nf_ref[...] + _mlp_refs(n_proc, dec)


def _tc_node_post(nf, n_enc, agg, nproc_ws, dec_ws):
    ws = list(nproc_ws) + list(dec_ws)
    return pl.pallas_call(
        _k3_body,
        out_shape=jax.ShapeDtypeStruct((N, D_NODE), jnp.float32),
    )(nf, n_enc, agg, *ws)


def kernel(edges, node_features, edge_features, params):
    e = edges[0]
    nf = node_features[0]
    ef = edge_features[0]

    enc_n = _mlp_params(params["node_encoder"])
    ee = params["edge_encoder"]
    pp = params["edge_processor"]
    wp_in = pp["in"]["W"]

    edge_ws = [
        ee["in"]["W"].astype(jnp.bfloat16), _bpair(ee["in"]["b"]),
        _bdiag(ee["blocks"][0]["a"]["W"]), _bpair(ee["blocks"][0]["a"]["b"]),
        _bdiag(ee["blocks"][0]["b"]["W"]), _bpair(ee["blocks"][0]["b"]["b"]),
        _bdiag(ee["out"]["W"]), _bpair(ee["out"]["b"]),
        _bdiag(wp_in[:W]), _bpair(pp["in"]["b"]),
        _bdiag(pp["blocks"][0]["a"]["W"]), _bpair(pp["blocks"][0]["a"]["b"]),
        _bdiag(pp["blocks"][0]["b"]["W"]), _bpair(pp["blocks"][0]["b"]["b"]),
        _bdiag(pp["out"]["W"]), _bpair(pp["out"]["b"]),
    ]

    np_ = params["node_processor"]
    wn_in = np_["in"]["W"]
    nproc_ws = [wn_in[:W], wn_in[W:], np_["in"]["b"].reshape(1, -1),
                np_["blocks"][0]["a"]["W"], np_["blocks"][0]["a"]["b"].reshape(1, -1),
                np_["blocks"][0]["b"]["W"], np_["blocks"][0]["b"]["b"].reshape(1, -1),
                np_["out"]["W"], np_["out"]["b"].reshape(1, -1)]
    dec_ws = _mlp_params(params["decoder"])

    n_enc, u_s, u_d = _tc_node_encoder(nf, enc_n, wp_in[W:2 * W], wp_in[2 * W:])
    gs2, gd2 = _sc_gather(u_s, u_d, e)
    e_proc2 = _tc_edge_mlp(ef.T, gs2, gd2, edge_ws)
    agg = _sc_scatter(e_proc2, e)
    out = _tc_node_post(nf, n_enc, agg, nproc_ws, dec_ws)
    return out[None]


# speedup vs baseline: 21.7749x; 1.0557x over previous
import functools

import jax
import jax.numpy as jnp
from jax import lax
from jax.experimental import pallas as pl
from jax.experimental.pallas import tpu as pltpu
from jax.experimental.pallas import tpu_sc as plsc

N = 10000
E = 320000
D_NODE = 128
D_EDGE = 16
W = 64

GW = 128
WPB = 2
PBLK = 3200
CHUNKS = 5
EC = E // CHUNKS
NSUB = 16
NCORE = 2
RPT = N // NSUB


def _f32dot(x, w):
    return jnp.dot(x, w, preferred_element_type=jnp.float32)


def _bdot(x, w):
    return jnp.dot(x.astype(jnp.bfloat16), w, preferred_element_type=jnp.float32)


def _mlp_refs(x, r):
    h = jnp.maximum(_f32dot(x, r[0][...]) + r[1][...], 0.0)
    h1 = jnp.maximum(_f32dot(h, r[2][...]) + r[3][...], 0.0)
    h2 = jnp.maximum(_f32dot(h1, r[4][...]) + r[5][...], 0.0)
    h = h + h2
    return _f32dot(h, r[6][...]) + r[7][...]


def _res_out(h, r):
    h1 = jnp.maximum(_f32dot(h, r[0][...]) + r[1][...], 0.0)
    h2 = jnp.maximum(_f32dot(h1, r[2][...]) + r[3][...], 0.0)
    h = h + h2
    return _f32dot(h, r[4][...]) + r[5][...]


def _mlp_params(p):
    return [p["in"]["W"], p["in"]["b"].reshape(1, -1),
            p["blocks"][0]["a"]["W"], p["blocks"][0]["a"]["b"].reshape(1, -1),
            p["blocks"][0]["b"]["W"], p["blocks"][0]["b"]["b"].reshape(1, -1),
            p["out"]["W"], p["out"]["b"].reshape(1, -1)]


def _bdiag(w):
    z = jnp.zeros(w.shape, w.dtype)
    return jnp.concatenate(
        [jnp.concatenate([w, z], axis=1), jnp.concatenate([z, w], axis=1)],
        axis=0).astype(jnp.bfloat16)


def _bpair(b):
    b = b.reshape(1, -1)
    return jnp.concatenate([b, b], axis=1)


def _k1_body(x_ref, wps_ref, wpd_ref, *refs):
    ne_ref, us_ref, ud_ref = refs[-3], refs[-2], refs[-1]
    n_enc = _mlp_refs(x_ref[...], refs[:-3])
    ne_ref[...] = n_enc
    us_ref[...] = _f32dot(n_enc, wps_ref[...])
    ud_ref[...] = _f32dot(n_enc, wpd_ref[...])


def _tc_node_encoder(nf, ws, wps, wpd):
    return pl.pallas_call(
        _k1_body,
        out_shape=[jax.ShapeDtypeStruct((N, W), jnp.float32),
                   jax.ShapeDtypeStruct((N, W), jnp.float32),
                   jax.ShapeDtypeStruct((N, W), jnp.float32)],
    )(nf, wps, wpd, *ws)


def _sc_gather(u_s, u_d, e, c):
    mesh = plsc.VectorSubcoreMesh(core_axis_name="c", subcore_axis_name="s")
    eb = GW * WPB
    hb = (EC // 2) // eb
    cb = c * (EC // eb)

    @functools.partial(
        pl.kernel,
        out_type=[jax.ShapeDtypeStruct((EC // 2, 2 * W), jnp.float32),
                  jax.ShapeDtypeStruct((EC // 2, 2 * W), jnp.float32)],
        mesh=mesh,
        compiler_params=pltpu.CompilerParams(use_tc_tiling_on_sc=False),
        scratch_types=[pltpu.SemaphoreType.DMA],
    )
    def k(us_hbm, ud_hbm, idx_hbm, os_hbm, od_hbm, sem):
        def body(i_vmem, os_vmem, od_vmem):
            cps = []
            for wnd in range(WPB):
                sl = pl.ds(wnd * GW, GW)
                cps.append(pltpu.async_copy(
                    us_hbm.at[i_vmem.at[0, sl]], os_vmem.at[sl], sem))
                cps.append(pltpu.async_copy(
                    ud_hbm.at[i_vmem.at[1, sl]], od_vmem.at[sl], sem))
            for cp in cps:
                cp.wait()

        pltpu.emit_pipeline(
            body,
            grid=(EC // eb,),
            in_specs=[pl.BlockSpec((2, eb), lambda i: (0, i + cb))],
            out_specs=[pl.BlockSpec((eb, W), lambda i: (i % hb, i // hb)),
                       pl.BlockSpec((eb, W), lambda i: (i % hb, i // hb))],
            core_axis_name=("c", "s"),
            dimension_semantics=(pltpu.PARALLEL,),
        )(idx_hbm, os_hbm, od_hbm)

    return k(u_s, u_d, e)


def _tdot(x, w):
    return lax.dot_general(x.astype(jnp.bfloat16), w,
                           (((0,), (0,)), ((), ())),
                           preferred_element_type=jnp.float32)


def _k2_body(eflo_ref, efhi_ref, gs_ref, gd_ref, *refs):
    o_ref = refs[-1]
    w1, b1, wa, ba, wb, bb, wo, bo = refs[0:8]
    wpe, bp, wpa, bpa, wpb, bpb, wpo, bpo = refs[8:16]
    h1 = jnp.concatenate([_tdot(eflo_ref[...], w1[...]),
                          _tdot(efhi_ref[...], w1[...])], axis=1)
    h = jnp.maximum(h1 + b1[...], 0.0)
    h1 = jnp.maximum(_bdot(h, wa[...]) + ba[...], 0.0)
    h2 = jnp.maximum(_bdot(h1, wb[...]) + bb[...], 0.0)
    h = h + h2
    e_enc = _bdot(h, wo[...]) + bo[...]
    pre = _bdot(e_enc, wpe[...]) + gs_ref[...] + gd_ref[...] + bp[...]
    h = jnp.maximum(pre, 0.0)
    h1 = jnp.maximum(_bdot(h, wpa[...]) + bpa[...], 0.0)
    h2 = jnp.maximum(_bdot(h1, wpb[...]) + bpb[...], 0.0)
    h = h + h2
    o_ref[...] = _bdot(h, wpo[...]) + bpo[...]


def _tc_edge_mlp(eft, gs2, gd2, ws, c):
    full = lambda a: pl.BlockSpec(a.shape, lambda i: (0,) * a.ndim)
    nb = (EC // 2) // PBLK
    clo = c * (EC // PBLK)
    return pl.pallas_call(
        _k2_body,
        grid=(nb,),
        in_specs=[
            pl.BlockSpec((D_EDGE, PBLK), lambda i: (0, i + clo)),
            pl.BlockSpec((D_EDGE, PBLK), lambda i: (0, i + clo + nb)),
            pl.BlockSpec((PBLK, 2 * W), lambda i: (i, 0)),
            pl.BlockSpec((PBLK, 2 * W), lambda i: (i, 0)),
        ] + [full(a) for a in ws],
        out_specs=pl.BlockSpec((PBLK, 2 * W), lambda i: (i, 0)),
        out_shape=jax.ShapeDtypeStruct((EC // 2, 2 * W), jnp.float32),
    )(eft, eft, gs2, gd2, *ws)


def _sc_scatter(e_proc2s, e):
    mesh = plsc.VectorSubcoreMesh(core_axis_name="c", subcore_axis_name="s")
    hw = (EC // 2) // GW

    @functools.partial(
        pl.kernel,
        out_type=jax.ShapeDtypeStruct((NCORE, N, W), jnp.float32),
        mesh=mesh,
        compiler_params=pltpu.CompilerParams(use_tc_tiling_on_sc=False),
        scratch_types=[
            pltpu.VMEM_SHARED((N, W), jnp.float32),
            pltpu.VMEM((RPT, W), jnp.float32),
        ],
    )
    def k(*refs):
        x_hbms = refs[:CHUNKS]
        idx_hbm = refs[CHUNKS]
        out_hbm = refs[CHUNKS + 1]
        acc, zbuf = refs[CHUNKS + 2], refs[CHUNKS + 3]
        cid = lax.axis_index("c")
        sid = lax.axis_index("s")

        @pl.loop(0, RPT)
        def _(r):
            for col in range(0, W, 16):
                zbuf[r, pl.ds(col, 16)] = jnp.zeros((16,), jnp.float32)

        pltpu.sync_copy(zbuf, acc.at[pl.ds(sid * RPT, RPT)])
        plsc.subcore_barrier()

        def body(x_vmem, i_vmem):
            pltpu.sync_copy(x_vmem, acc.at[i_vmem.at[0]], add=True)

        for c in range(CHUNKS):
            cw = c * (EC // GW)
            pltpu.emit_pipeline(
                body,
                grid=(EC // GW,),
                in_specs=[
                    pl.BlockSpec((GW, W), lambda i: (i % hw, i // hw)),
                    pl.BlockSpec((1, GW), lambda i, cw=cw: (0, i + cw)),
                ],
                out_specs=[],
                core_axis_name=("c", "s"),
                dimension_semantics=(pltpu.PARALLEL,),
            )(x_hbms[c], idx_hbm)

        plsc.subcore_barrier()
        pltpu.sync_copy(acc.at[pl.ds(sid * RPT, RPT)],
                        out_hbm.at[cid, pl.ds(sid * RPT, RPT)])

    return k(*e_proc2s, e)


def _k3_body(nf_ref, ne_ref, agg_ref, *refs):
    o_ref = refs[-1]
    wna, wnb, bn = refs[0], refs[1], refs[2]
    nproc_rest = refs[3:9]
    dec = refs[9:17]
    a = agg_ref[0] + agg_ref[1]
    pre = _f32dot(ne_ref[...], wna[...]) + _f32dot(a, wnb[...]) + bn[...]
    n_proc = _res_out(jnp.maximum(pre, 0.0), nproc_rest)
    o_ref[...] = nf_ref[...] + _mlp_refs(n_proc, dec)


def _tc_node_post(nf, n_enc, agg, nproc_ws, dec_ws):
    ws = list(nproc_ws) + list(dec_ws)
    return pl.pallas_call(
        _k3_body,
        out_shape=jax.ShapeDtypeStruct((N, D_NODE), jnp.float32),
    )(nf, n_enc, agg, *ws)


def kernel(edges, node_features, edge_features, params):
    e = edges[0]
    nf = node_features[0]
    ef = edge_features[0]

    enc_n = _mlp_params(params["node_encoder"])
    ee = params["edge_encoder"]
    pp = params["edge_processor"]
    wp_in = pp["in"]["W"]

    edge_ws = [
        ee["in"]["W"].astype(jnp.bfloat16), _bpair(ee["in"]["b"]),
        _bdiag(ee["blocks"][0]["a"]["W"]), _bpair(ee["blocks"][0]["a"]["b"]),
        _bdiag(ee["blocks"][0]["b"]["W"]), _bpair(ee["blocks"][0]["b"]["b"]),
        _bdiag(ee["out"]["W"]), _bpair(ee["out"]["b"]),
        _bdiag(wp_in[:W]), _bpair(pp["in"]["b"]),
        _bdiag(pp["blocks"][0]["a"]["W"]), _bpair(pp["blocks"][0]["a"]["b"]),
        _bdiag(pp["blocks"][0]["b"]["W"]), _bpair(pp["blocks"][0]["b"]["b"]),
        _bdiag(pp["out"]["W"]), _bpair(pp["out"]["b"]),
    ]

    np_ = params["node_processor"]
    wn_in = np_["in"]["W"]
    nproc_ws = [wn_in[:W], wn_in[W:], np_["in"]["b"].reshape(1, -1),
                np_["blocks"][0]["a"]["W"], np_["blocks"][0]["a"]["b"].reshape(1, -1),
                np_["blocks"][0]["b"]["W"], np_["blocks"][0]["b"]["b"].reshape(1, -1),
                np_["out"]["W"], np_["out"]["b"].reshape(1, -1)]
    dec_ws = _mlp_params(params["decoder"])

    n_enc, u_s, u_d = _tc_node_encoder(nf, enc_n, wp_in[W:2 * W], wp_in[2 * W:])
    eft = ef.T
    e_proc2s = []
    for c in range(CHUNKS):
        gs2, gd2 = _sc_gather(u_s, u_d, e, c)
        e_proc2s.append(_tc_edge_mlp(eft, gs2, gd2, edge_ws, c))
    agg = _sc_scatter(e_proc2s, e)
    out = _tc_node_post(nf, n_enc, agg, nproc_ws, dec_ws)
    return out[None]
